# baseline (speedup 1.0000x reference)
import math

import jax
import jax.numpy as jnp
from jax import lax
from jax.experimental import pallas as pl
from jax.experimental.pallas import tpu as pltpu
from jax.experimental.pallas import tpu_sc as plsc

VOCAB = 1000000
D_MODEL = 64
BATCH = 4096
HIST = 200
SCALE = math.sqrt(D_MODEL)

NC = 2
NS = 16
NW = NC * NS
LANES = 16
NVEC = D_MODEL // LANES
BBLK = BATCH // NW

VSLAB = 128
NFULL = VOCAB // VSLAB
VTAIL = VOCAB - NFULL * VSLAB
EXTRA_W = NFULL - NW * 244

M = 4
L = 2
NOUTER = HIST // M


def _prep_body(wt_hbm, tail_hbm, t_out, inb, outb, rsem, wsem):
    wid = lax.axis_index("s") * NC + lax.axis_index("c")
    iotas = [lax.iota(jnp.int32, LANES) + c * LANES for c in range(NVEC)]

    def v0_of(k):
        return (wid + k * NW) * VSLAB

    def read(k, s):
        return pltpu.make_async_copy(
            wt_hbm.at[:, pl.ds(v0_of(k), VSLAB)], inb.at[s], rsem.at[s]
        )

    def write(k, s):
        return pltpu.make_async_copy(
            outb.at[s], t_out.at[pl.ds(v0_of(k), VSLAB)], wsem.at[s]
        )

    def transpose(s):
        def d_step(d, c2):
            col = jnp.full((LANES,), d, dtype=jnp.int32)
            for jb in range(VSLAB // LANES):
                vals = inb[s, d, pl.ds(jb * LANES, LANES)] * SCALE
                plsc.store_scatter(
                    outb.at[s], [iotas[0] + jb * LANES, col], vals
                )
            return c2

        lax.fori_loop(0, D_MODEL, d_step, 0)

    for s in range(2):
        read(s, s).start()

    def turn(k, s, issue=True, wait_old=True):
        read(k, s).wait()
        if wait_old:
            write(k - 2, s).wait()
        transpose(s)
        if issue:
            read(k + 2, s).start()
        write(k, s).start()

    turn(0, 0, wait_old=False)
    turn(1, 1, wait_old=False)

    def outer(o, carry):
        for b in range(2):
            turn(o * 2 + b, b)
        return carry

    lax.fori_loop(1, 121, outer, 0)

    def issue_extra():
        @pl.when(wid < EXTRA_W)
        def _():
            read(244, 0).start()

    turn(242, 0, issue=False)
    issue_extra()
    turn(243, 1, issue=False)

    @pl.when(wid < EXTRA_W)
    def _():
        turn(244, 0, issue=False)
        write(244, 0).wait()

    @pl.when(wid >= EXTRA_W)
    def _():
        write(242, 0).wait()

    write(243, 1).wait()

    @pl.when(wid == 0)
    def _():
        v0 = NFULL * VSLAB
        pltpu.sync_copy(tail_hbm, inb.at[0])
        transpose(0)
        pltpu.sync_copy(outb.at[0], t_out.at[pl.ds(v0, VSLAB)])


def _lookup_body(xt_hbm, t_hbm, out_hbm, idx_v, gbuf, tbuf, gsem, ssem):
    wid = lax.axis_index("s") * NC + lax.axis_index("c")
    b0 = wid * BBLK
    pltpu.sync_copy(xt_hbm.at[:, pl.ds(b0, BBLK)], idx_v)
    iotas = [lax.iota(jnp.int32, LANES) + c * LANES for c in range(NVEC)]

    def gather(h, slot):
        return pltpu.make_async_copy(
            t_hbm.at[idx_v.at[h]], gbuf.at[slot], gsem.at[slot]
        )

    def scatter(h, slot):
        return pltpu.make_async_copy(
            tbuf.at[slot], out_hbm.at[h, :, pl.ds(b0, BBLK)], ssem.at[slot]
        )

    def transpose(gs, ts):
        def b_step(b, c2):
            colb = jnp.full((LANES,), b, dtype=jnp.int32)
            for c in range(NVEC):
                vals = gbuf[gs, b, pl.ds(c * LANES, LANES)]
                plsc.store_scatter(tbuf.at[ts], [iotas[c], colb], vals)
            return c2

        lax.fori_loop(0, BBLK, b_step, 0)

    for s in range(L):
        gather(s, s).start()

    def turn(h, b, issue, wait_old):
        gather(h, b).wait()
        if issue:
            gather(h + L, (b + L) % M).start()
        if wait_old:
            scatter(h - 2, b % 2).wait()
        transpose(b, b % 2)
        scatter(h, b % 2).start()

    for b in range(M):
        turn(b, b, issue=True, wait_old=(b >= 2))

    def outer(o, carry):
        for b in range(M):
            turn(o * M + b, b, issue=True, wait_old=True)
        return carry

    lax.fori_loop(1, NOUTER - 1, outer, 0)

    for b in range(M):
        h = (NOUTER - 1) * M + b
        turn(h, b, issue=(b < M - L), wait_old=True)

    scatter(HIST - 2, 0).wait()
    scatter(HIST - 1, 1).wait()


@jax.jit
def _embed(xt, wt):
    tail = jnp.pad(wt[:, NFULL * VSLAB:], ((0, 0), (0, VSLAB - VTAIL)))
    mesh = plsc.VectorSubcoreMesh(core_axis_name="c", subcore_axis_name="s")
    prep = pl.kernel(
        _prep_body,
        out_type=jax.ShapeDtypeStruct((NFULL * VSLAB + VSLAB, 2 * D_MODEL), jnp.float32),
        mesh=mesh,
        scratch_types=[
            pltpu.VMEM((2, D_MODEL, VSLAB), jnp.float32),
            pltpu.VMEM((2, VSLAB, 2 * D_MODEL), jnp.float32),
            pltpu.SemaphoreType.DMA((2,)),
            pltpu.SemaphoreType.DMA((2,)),
        ],
        compiler_params=pltpu.CompilerParams(use_tc_tiling_on_sc=True, needs_layout_passes=False),
    )
    t = prep(wt, tail)
    look = pl.kernel(
        _lookup_body,
        out_type=jax.ShapeDtypeStruct((HIST, D_MODEL, BATCH), jnp.float32),
        mesh=mesh,
        scratch_types=[
            pltpu.VMEM((HIST, BBLK), jnp.int32),
            pltpu.VMEM((M, BBLK, 2 * D_MODEL), jnp.float32),
            pltpu.VMEM((2, D_MODEL, BBLK), jnp.float32),
            pltpu.SemaphoreType.DMA((M,)),
            pltpu.SemaphoreType.DMA((2,)),
        ],
        compiler_params=pltpu.CompilerParams(use_tc_tiling_on_sc=True, needs_layout_passes=False),
    )
    return look(xt, t)


def kernel(x, weight):
    ol = _embed(x.T, weight.T)
    return jnp.transpose(ol, (2, 0, 1))


# speedup vs baseline: 1.6620x; 1.6620x over previous
import math

import jax
import jax.numpy as jnp
from jax import lax
from jax.experimental import pallas as pl
from jax.experimental.pallas import tpu as pltpu
from jax.experimental.pallas import tpu_sc as plsc

VOCAB = 1000000
D_MODEL = 64
BATCH = 4096
HIST = 200
SCALE = math.sqrt(D_MODEL)

NC = 2
NS = 16
NW = NC * NS
LANES = 16
NVEC = D_MODEL // LANES

B_TOTAL = BATCH * HIST
ROWS_PER_W = B_TOTAL // NW
CHUNK = 128
NCHUNK = ROWS_PER_W // CHUNK
M = 4
L = 2
NOUTER = NCHUNK // M


def _lookup_body(x_hbm, t_hbm, out_hbm, idx_v, gbuf, obuf, gsem, ssem):
    wid = lax.axis_index("s") * NC + lax.axis_index("c")
    pltpu.sync_copy(x_hbm.at[wid], idx_v)
    base = wid * ROWS_PER_W
    out_flat = out_hbm.reshape(B_TOTAL, D_MODEL)

    def gather(g, slot):
        return pltpu.make_async_copy(
            t_hbm.at[idx_v.at[g]], gbuf.at[slot], gsem.at[slot]
        )

    def scatter(g, slot):
        return pltpu.make_async_copy(
            obuf.at[slot], out_flat.at[pl.ds(base + g * CHUNK, CHUNK)], ssem.at[slot]
        )

    def compact(gs, os_):
        def row_step(r, c2):
            for c in range(NVEC):
                sl = pl.ds(c * LANES, LANES)
                obuf[os_, r, sl] = gbuf[gs, r, sl]
            return c2

        lax.fori_loop(0, CHUNK, row_step, 0, unroll=2)

    for s in range(L):
        gather(s, s).start()

    def turn(g, b, issue, wait_old):
        gather(g, b).wait()
        if issue:
            gather(g + L, (b + L) % M).start()
        if wait_old:
            scatter(g - 2, b % 2).wait()
        compact(b, b % 2)
        scatter(g, b % 2).start()

    for b in range(M):
        turn(b, b, issue=True, wait_old=(b >= 2))

    def outer(o, carry):
        for b in range(M):
            turn(o * M + b, b, issue=True, wait_old=True)
        return carry

    lax.fori_loop(1, NOUTER - 1, outer, 0)

    for b in range(M):
        g = (NOUTER - 1) * M + b
        turn(g, b, issue=(b < M - L), wait_old=True)

    scatter(NCHUNK - 2, 0).wait()
    scatter(NCHUNK - 1, 1).wait()


@jax.jit
def _embed(x_grouped, weight):
    t = jnp.pad(weight * SCALE, ((0, 0), (0, D_MODEL)))
    mesh = plsc.VectorSubcoreMesh(core_axis_name="c", subcore_axis_name="s")
    look = pl.kernel(
        _lookup_body,
        out_type=jax.ShapeDtypeStruct((BATCH, HIST, D_MODEL), jnp.float32),
        mesh=mesh,
        scratch_types=[
            pltpu.VMEM((NCHUNK, CHUNK), jnp.int32),
            pltpu.VMEM((M, CHUNK, 2 * D_MODEL), jnp.float32),
            pltpu.VMEM((2, CHUNK, D_MODEL), jnp.float32),
            pltpu.SemaphoreType.DMA((M,)),
            pltpu.SemaphoreType.DMA((2,)),
        ],
        compiler_params=pltpu.CompilerParams(use_tc_tiling_on_sc=True),
    )
    return look(x_grouped, t)


def kernel(x, weight):
    x_grouped = x.reshape(NW, NCHUNK, CHUNK)
    return _embed(x_grouped, weight)


# speedup vs baseline: 1.9826x; 1.1929x over previous
import math

import jax
import jax.numpy as jnp
from jax import lax
from jax.experimental import pallas as pl
from jax.experimental.pallas import tpu as pltpu
from jax.experimental.pallas import tpu_sc as plsc

VOCAB = 1000000
D_MODEL = 64
BATCH = 4096
HIST = 200
SCALE = math.sqrt(D_MODEL)

NC = 2
NS = 16
NW = NC * NS
LANES = 16
NVEC = D_MODEL // LANES

B_TOTAL = BATCH * HIST
ROWS_PER_W = B_TOTAL // NW
CHUNK = 128
NCHUNK = ROWS_PER_W // CHUNK

SLAB = 160
NSLAB = VOCAB // SLAB
FULL_K = 195
EXTRA_W = NSLAB - NW * FULL_K

M = 4
L = 2
NOUTER = NCHUNK // M


def _prep_body(table_hbm, t_out, inb, outb, rsem, wsem):
    wid = lax.axis_index("s") * NC + lax.axis_index("c")

    def slab_of(k):
        return wid + k * NW

    def read(k, s):
        rows = pl.ds(slab_of(k) * SLAB, SLAB)
        return pltpu.make_async_copy(table_hbm.at[rows], inb.at[s], rsem.at[s])

    def write(k, s):
        rows = pl.ds(slab_of(k) * SLAB, SLAB)
        return pltpu.make_async_copy(outb.at[s], t_out.at[rows], wsem.at[s])

    def scale_copy(s):
        def row_step(r, c2):
            for c in range(NVEC):
                sl = pl.ds(c * LANES, LANES)
                outb[s, r, sl] = inb[s, r, sl] * SCALE
            return c2

        lax.fori_loop(0, SLAB, row_step, 0, unroll=2)

    for s in range(3):
        read(s, s).start()

    def turn(k, s, issue_guard, wait_old=True):
        read(k, s).wait()
        if wait_old:
            write(k - 3, s).wait()
        scale_copy(s)
        kn = k + 3
        if issue_guard is None:
            read(kn, s).start()
        elif issue_guard:
            @pl.when(wid < EXTRA_W)
            def _():
                read(kn, s).start()
        write(k, s).start()

    for b in range(3):
        turn(b, b, None, wait_old=False)

    def outer(o, carry):
        for b in range(3):
            turn(o * 3 + b, b, None)
        return carry

    lax.fori_loop(1, 64, outer, 0)

    turn(192, 0, True)
    turn(193, 1, False)
    turn(194, 2, False)

    @pl.when(wid < EXTRA_W)
    def _():
        turn(195, 0, False)
        write(195, 0).wait()

    @pl.when(wid >= EXTRA_W)
    def _():
        write(192, 0).wait()

    write(193, 1).wait()
    write(194, 2).wait()


def _lookup_body(x_hbm, t_hbm, out_hbm, idx_v, gbuf, obuf, gsem, ssem):
    wid = lax.axis_index("s") * NC + lax.axis_index("c")
    pltpu.sync_copy(x_hbm.at[wid], idx_v)
    base = wid * ROWS_PER_W

    def gather(g, slot):
        return pltpu.make_async_copy(
            t_hbm.at[idx_v.at[g]], gbuf.at[slot], gsem.at[slot]
        )

    def scatter(g, slot):
        return pltpu.make_async_copy(
            obuf.at[slot], out_hbm.at[pl.ds(base + g * CHUNK, CHUNK)], ssem.at[slot]
        )

    def compact(gs, os_):
        def row_step(r, c2):
            for c in range(NVEC):
                sl = pl.ds(c * LANES, LANES)
                obuf[os_, r, sl] = gbuf[gs, r, sl]
            return c2

        lax.fori_loop(0, CHUNK, row_step, 0, unroll=2)

    for s in range(L):
        gather(s, s).start()

    def turn(g, b, issue, wait_old):
        gather(g, b).wait()
        if issue:
            gather(g + L, (b + L) % M).start()
        if wait_old:
            scatter(g - 2, b % 2).wait()
        compact(b, b % 2)
        scatter(g, b % 2).start()

    for b in range(M):
        turn(b, b, issue=True, wait_old=(b >= 2))

    def outer(o, carry):
        for b in range(M):
            turn(o * M + b, b, issue=True, wait_old=True)
        return carry

    lax.fori_loop(1, NOUTER - 1, outer, 0)

    for b in range(M):
        g = (NOUTER - 1) * M + b
        turn(g, b, issue=(b < M - L), wait_old=True)

    scatter(NCHUNK - 2, 0).wait()
    scatter(NCHUNK - 1, 1).wait()


@jax.jit
def _embed(x_grouped, weight):
    mesh = plsc.VectorSubcoreMesh(core_axis_name="c", subcore_axis_name="s")
    prep = pl.kernel(
        _prep_body,
        out_type=jax.ShapeDtypeStruct((VOCAB, 2 * D_MODEL), jnp.float32),
        mesh=mesh,
        scratch_types=[
            pltpu.VMEM((3, SLAB, D_MODEL), jnp.float32),
            pltpu.VMEM((3, SLAB, 2 * D_MODEL), jnp.float32),
            pltpu.SemaphoreType.DMA((3,)),
            pltpu.SemaphoreType.DMA((3,)),
        ],
        compiler_params=pltpu.CompilerParams(use_tc_tiling_on_sc=True),
    )
    t = prep(weight)
    look = pl.kernel(
        _lookup_body,
        out_type=jax.ShapeDtypeStruct((B_TOTAL, D_MODEL), jnp.float32),
        mesh=mesh,
        scratch_types=[
            pltpu.VMEM((NCHUNK, CHUNK), jnp.int32),
            pltpu.VMEM((M, CHUNK, 2 * D_MODEL), jnp.float32),
            pltpu.VMEM((2, CHUNK, D_MODEL), jnp.float32),
            pltpu.SemaphoreType.DMA((M,)),
            pltpu.SemaphoreType.DMA((2,)),
        ],
        compiler_params=pltpu.CompilerParams(use_tc_tiling_on_sc=True),
    )
    out = look(x_grouped, t)
    return out.reshape(BATCH, HIST, D_MODEL)


def kernel(x, weight):
    x_grouped = x.reshape(NW, NCHUNK, CHUNK)
    return _embed(x_grouped, weight)


# speedup vs baseline: 1.9840x; 1.0007x over previous
import math

import jax
import jax.numpy as jnp
from jax import lax
from jax.experimental import pallas as pl
from jax.experimental.pallas import tpu as pltpu
from jax.experimental.pallas import tpu_sc as plsc

VOCAB = 1000000
D_MODEL = 64
BATCH = 4096
HIST = 200
SCALE = math.sqrt(D_MODEL)

NC = 2
NS = 16
NW = NC * NS
LANES = 16
NVEC = D_MODEL // LANES

B_TOTAL = BATCH * HIST
ROWS_PER_W = B_TOTAL // NW
CHUNK = 128
NCHUNK = ROWS_PER_W // CHUNK

SLAB = 160
NSLAB = VOCAB // SLAB
FULL_K = 195
EXTRA_W = NSLAB - NW * FULL_K

M = 4
L = 3
NOUTER = NCHUNK // M


def _prep_body(table_hbm, t_out, inb, outb, rsem, wsem):
    wid = lax.axis_index("s") * NC + lax.axis_index("c")

    def slab_of(k):
        return wid + k * NW

    def read(k, s):
        rows = pl.ds(slab_of(k) * SLAB, SLAB)
        return pltpu.make_async_copy(table_hbm.at[rows], inb.at[s], rsem.at[s])

    def write(k, s):
        rows = pl.ds(slab_of(k) * SLAB, SLAB)
        return pltpu.make_async_copy(outb.at[s], t_out.at[rows], wsem.at[s])

    def scale_copy(s):
        def row_step(r, c2):
            for c in range(NVEC):
                sl = pl.ds(c * LANES, LANES)
                outb[s, r, sl] = inb[s, r, sl] * SCALE
            return c2

        lax.fori_loop(0, SLAB, row_step, 0, unroll=2)

    for s in range(3):
        read(s, s).start()

    def turn(k, s, issue_guard, wait_old=True):
        read(k, s).wait()
        if wait_old:
            write(k - 3, s).wait()
        scale_copy(s)
        kn = k + 3
        if issue_guard is None:
            read(kn, s).start()
        elif issue_guard:
            @pl.when(wid < EXTRA_W)
            def _():
                read(kn, s).start()
        write(k, s).start()

    for b in range(3):
        turn(b, b, None, wait_old=False)

    def outer(o, carry):
        for b in range(3):
            turn(o * 3 + b, b, None)
        return carry

    lax.fori_loop(1, 64, outer, 0)

    turn(192, 0, True)
    turn(193, 1, False)
    turn(194, 2, False)

    @pl.when(wid < EXTRA_W)
    def _():
        turn(195, 0, False)
        write(195, 0).wait()

    @pl.when(wid >= EXTRA_W)
    def _():
        write(192, 0).wait()

    write(193, 1).wait()
    write(194, 2).wait()


def _lookup_body(x_hbm, t_hbm, out_hbm, idx_v, gbuf, obuf, gsem, ssem):
    wid = lax.axis_index("s") * NC + lax.axis_index("c")
    pltpu.sync_copy(x_hbm.at[wid], idx_v)
    base = wid * ROWS_PER_W

    def gather(g, slot):
        return pltpu.make_async_copy(
            t_hbm.at[idx_v.at[g]], gbuf.at[slot], gsem.at[slot]
        )

    def scatter(g, slot):
        return pltpu.make_async_copy(
            obuf.at[slot], out_hbm.at[pl.ds(base + g * CHUNK, CHUNK)], ssem.at[slot]
        )

    def compact(gs, os_):
        def row_step(r, c2):
            for c in range(NVEC):
                sl = pl.ds(c * LANES, LANES)
                obuf[os_, r, sl] = gbuf[gs, r, sl]
            return c2

        lax.fori_loop(0, CHUNK, row_step, 0, unroll=2)

    for s in range(L):
        gather(s, s).start()

    def turn(g, b, issue, wait_old):
        gather(g, b).wait()
        if issue:
            gather(g + L, (b + L) % M).start()
        if wait_old:
            scatter(g - 2, b % 2).wait()
        compact(b, b % 2)
        scatter(g, b % 2).start()

    for b in range(M):
        turn(b, b, issue=True, wait_old=(b >= 2))

    def outer(o, carry):
        for b in range(M):
            turn(o * M + b, b, issue=True, wait_old=True)
        return carry

    lax.fori_loop(1, NOUTER - 1, outer, 0)

    for b in range(M):
        g = (NOUTER - 1) * M + b
        turn(g, b, issue=(b < M - L), wait_old=True)

    scatter(NCHUNK - 2, 0).wait()
    scatter(NCHUNK - 1, 1).wait()


@jax.jit
def _embed(x_grouped, weight):
    mesh = plsc.VectorSubcoreMesh(core_axis_name="c", subcore_axis_name="s")
    prep = pl.kernel(
        _prep_body,
        out_type=jax.ShapeDtypeStruct((VOCAB, 2 * D_MODEL), jnp.float32),
        mesh=mesh,
        scratch_types=[
            pltpu.VMEM((3, SLAB, D_MODEL), jnp.float32),
            pltpu.VMEM((3, SLAB, 2 * D_MODEL), jnp.float32),
            pltpu.SemaphoreType.DMA((3,)),
            pltpu.SemaphoreType.DMA((3,)),
        ],
        compiler_params=pltpu.CompilerParams(use_tc_tiling_on_sc=True),
    )
    t = prep(weight)
    look = pl.kernel(
        _lookup_body,
        out_type=jax.ShapeDtypeStruct((B_TOTAL, D_MODEL), jnp.float32),
        mesh=mesh,
        scratch_types=[
            pltpu.VMEM((NCHUNK, CHUNK), jnp.int32),
            pltpu.VMEM((M, CHUNK, 2 * D_MODEL), jnp.float32),
            pltpu.VMEM((2, CHUNK, D_MODEL), jnp.float32),
            pltpu.SemaphoreType.DMA((M,)),
            pltpu.SemaphoreType.DMA((2,)),
        ],
        compiler_params=pltpu.CompilerParams(use_tc_tiling_on_sc=True),
    )
    out = look(x_grouped, t)
    return out.reshape(BATCH, HIST, D_MODEL)


def kernel(x, weight):
    x_grouped = x.reshape(NW, NCHUNK, CHUNK)
    return _embed(x_grouped, weight)
